# Initial kernel scaffold; baseline (speedup 1.0000x reference)
import jax
import jax.numpy as jnp
from jax.experimental import pallas as pl


def kernel(x, tables):
    raise NotImplementedError("write your pallas kernel here")


# speedup vs baseline: 18.3212x; 18.3212x over previous
import functools

import jax
import jax.numpy as jnp
from jax import lax
from jax.experimental import pallas as pl
from jax.experimental.pallas import tpu as pltpu
from jax.experimental.pallas import tpu_sc as plsc

N_POINTS = 262144
N_LEVELS = 16
N_FEATS = 2
TS = 2 ** 19
MASK = TS - 1
PR1 = -1640531535
PR2 = 805459861
NC, NS = 2, 16
NW = NC * NS
PPW = N_POINTS // NW
CHUNK = 512
NCHUNK = PPW // CHUNK
G = CHUNK // 16
OUT_D = N_LEVELS * N_FEATS


def _sc_embed(xt, ftab, gs):
    mesh = plsc.VectorSubcoreMesh(
        core_axis_name="c", subcore_axis_name="s",
        num_cores=NC, num_subcores=NS)

    @functools.partial(
        pl.kernel,
        out_type=jax.ShapeDtypeStruct((N_POINTS, OUT_D), jnp.float32),
        mesh=mesh,
        compiler_params=pltpu.CompilerParams(
            needs_layout_passes=False, use_tc_tiling_on_sc=False),
        scratch_types=[
            pltpu.VMEM((CHUNK,), jnp.float32),
            pltpu.VMEM((CHUNK,), jnp.float32),
            pltpu.VMEM((CHUNK,), jnp.float32),
            pltpu.VMEM((CHUNK * 8,), jnp.int32),
            pltpu.VMEM((CHUNK * 8,), jnp.int32),
            pltpu.VMEM((CHUNK * 8, 8), jnp.float32),
            pltpu.VMEM((CHUNK * 8, 8), jnp.float32),
            pltpu.VMEM((CHUNK, OUT_D), jnp.float32),
            pltpu.VMEM((16, 16), jnp.float32),
            pltpu.SemaphoreType.DMA,
            pltpu.SemaphoreType.DMA,
        ],
    )
    def body(xt_hbm, ftab_hbm, gs_hbm, out_hbm,
             xs, ys, zs, ia, ib, ra, rb, ob, gsv, sa, sb):
        wid = lax.axis_index("s") * NC + lax.axis_index("c")
        base_w = wid * PPW
        pltpu.sync_copy(gs_hbm, gsv)
        iota = lax.iota(jnp.int32, 16)
        lane8 = iota * 8
        lane8c = [lane8 + c for c in range(8)]

        def gs_bcast(l):
            return gsv[l]
        idxbufs = [ia, ib]
        rowbufs = [ra, rb]
        sems = [sa, sb]
        zero_f = jnp.float32(0.0)
        one_f = jnp.float32(1.0)

        def load_xyz(g):
            x = xs[pl.ds(g * 16, 16)]
            y = ys[pl.ds(g * 16, 16)]
            z = zs[pl.ds(g * 16, 16)]
            return x, y, z

        def corner_idx(x, y, z, gsl):
            xc = jnp.minimum(jnp.maximum(x, zero_f), one_f)
            yc = jnp.minimum(jnp.maximum(y, zero_f), one_f)
            zc = jnp.minimum(jnp.maximum(z, zero_f), one_f)
            xi = (xc / gsl).astype(jnp.int32)
            yi = (yc / gsl).astype(jnp.int32)
            zi = (zc / gsl).astype(jnp.int32)
            return xi, yi, zi

        def phase_a(l, g):
            x, y, z = load_xyz(g)
            xi, yi, zi = corner_idx(x, y, z, gs_bcast(l))
            hx = (xi, xi + 1)
            y0 = yi * jnp.int32(PR1)
            hy = (y0, y0 + jnp.int32(PR1))
            z0 = zi * jnp.int32(PR2)
            hz = (z0, z0 + jnp.int32(PR2))
            pos0 = g * 128
            loff = l * TS
            dst = idxbufs[l % 2]
            for c in range(8):
                i, j, k = c >> 2, (c >> 1) & 1, c & 1
                h = ((hx[i] ^ hy[j] ^ hz[k]) & MASK) + loff
                plsc.store_scatter(dst, [pos0 + lane8c[c]], h)

        def phase_c(l, g):
            x, y, z = load_xyz(g)
            gsl = gs_bcast(l)
            xi, yi, zi = corner_idx(x, y, z, gsl)
            wx = (x - xi.astype(jnp.float32) * gsl) / gsl
            wy = (y - yi.astype(jnp.float32) * gsl) / gsl
            wz = (z - zi.astype(jnp.float32) * gsl) / gsl
            pos0 = g * 128
            rows = rowbufs[l % 2]
            prow = g * 16 + iota
            for f in range(N_FEATS):
                fv = jnp.full((16,), f, jnp.int32)
                e = [plsc.load_gather(rows, [pos0 + lane8c[c], fv])
                     for c in range(8)]
                c00 = e[0] * (one_f - wx) + e[4] * wx
                c01 = e[1] * (one_f - wx) + e[5] * wx
                c10 = e[2] * (one_f - wx) + e[6] * wx
                c11 = e[3] * (one_f - wx) + e[7] * wx
                c0 = c00 * (one_f - wy) + c10 * wy
                c1 = c01 * (one_f - wy) + c11 * wy
                r = c0 * (one_f - wz) + c1 * wz
                plsc.store_scatter(ob, [prow, jnp.full((16,), 2 * l + f, jnp.int32)], r)

        def chunk_body(ci, carry):
            cb = base_w + ci * CHUNK
            pltpu.sync_copy(xt_hbm.at[pl.ds(cb, CHUNK)], xs)
            pltpu.sync_copy(xt_hbm.at[pl.ds(N_POINTS + cb, CHUNK)], ys)
            pltpu.sync_copy(xt_hbm.at[pl.ds(2 * N_POINTS + cb, CHUNK)], zs)
            lax.fori_loop(0, G, lambda g, _: phase_a(0, g), None)
            cps = [None] * N_LEVELS
            cps[0] = pltpu.async_copy(ftab_hbm.at[idxbufs[0]], rowbufs[0], sems[0])
            for l in range(N_LEVELS):
                if l + 1 < N_LEVELS:
                    lax.fori_loop(0, G, lambda g, _, l=l: phase_a(l + 1, g), None)
                    cps[l + 1] = pltpu.async_copy(
                        ftab_hbm.at[idxbufs[(l + 1) % 2]],
                        rowbufs[(l + 1) % 2], sems[(l + 1) % 2])
                cps[l].wait()
                lax.fori_loop(0, G, lambda g, _, l=l: phase_c(l, g), None)
            pltpu.sync_copy(ob, out_hbm.at[pl.ds(cb, CHUNK)])
            return carry

        lax.fori_loop(0, NCHUNK, chunk_body, None)

    return body(xt, ftab, gs)


def kernel(x, tables):
    b = jnp.exp((jnp.log(jnp.float32(512.0)) - jnp.log(jnp.float32(16.0)))
                / (N_LEVELS - 1))
    res = jnp.stack([jnp.floor(jnp.float32(16.0) * b ** i)
                     for i in range(N_LEVELS)])
    gs = (jnp.float32(1.0) - jnp.float32(0.0)) / res
    gs_rep = jnp.tile(gs[:, None], (1, 16))
    xt = x.T.reshape(3 * N_POINTS)
    ftab = tables.reshape(N_LEVELS * TS, N_FEATS)
    ftab = jnp.concatenate(
        [ftab, jnp.zeros((N_LEVELS * TS, 6), jnp.float32)], axis=1)
    out = _sc_embed(xt, ftab, gs_rep)
    keep_mask = x == jnp.maximum(jnp.minimum(x, 1.0), 0.0)
    keep_all = jnp.sum(keep_mask, axis=-1) == keep_mask.shape[-1]
    return out, keep_all


# speedup vs baseline: 22.3885x; 1.2220x over previous
import functools

import jax
import jax.numpy as jnp
from jax import lax
from jax.experimental import pallas as pl
from jax.experimental.pallas import tpu as pltpu
from jax.experimental.pallas import tpu_sc as plsc

N_POINTS = 262144
N_LEVELS = 16
N_FEATS = 2
TS = 2 ** 19
MASK = TS - 1
PR1 = -1640531535
PR2 = 805459861
NC, NS = 2, 16
NW = NC * NS
PPW = N_POINTS // NW
CHUNK = 512
NCHUNK = PPW // CHUNK
G = CHUNK // 16
OUT_D = N_LEVELS * N_FEATS


def _sc_embed(xt, ftab, gs):
    mesh = plsc.VectorSubcoreMesh(
        core_axis_name="c", subcore_axis_name="s",
        num_cores=NC, num_subcores=NS)

    @functools.partial(
        pl.kernel,
        out_type=jax.ShapeDtypeStruct((N_POINTS, OUT_D), jnp.float32),
        mesh=mesh,
        compiler_params=pltpu.CompilerParams(
            needs_layout_passes=False, use_tc_tiling_on_sc=False),
        scratch_types=[
            pltpu.VMEM((CHUNK * 3,), jnp.float32),
            pltpu.VMEM((CHUNK * 8,), jnp.int32),
            pltpu.VMEM((CHUNK * 8,), jnp.int32),
            pltpu.VMEM((CHUNK * 8, 8), jnp.float32),
            pltpu.VMEM((CHUNK * 8, 8), jnp.float32),
            pltpu.VMEM((CHUNK, OUT_D), jnp.float32),
            pltpu.VMEM((16, 16), jnp.float32),
            pltpu.SemaphoreType.DMA,
            pltpu.SemaphoreType.DMA,
        ],
    )
    def body(xt_hbm, ftab_hbm, gs_hbm, out_hbm,
             xyz, ia, ib, ra, rb, ob, gsv, sa, sb):
        wid = lax.axis_index("s") * NC + lax.axis_index("c")
        base_w = wid * PPW
        pltpu.sync_copy(gs_hbm, gsv)
        iota = lax.iota(jnp.int32, 16)
        lane8 = iota * 8
        lane8c = [lane8 + c for c in range(8)]
        iota3 = iota * 3

        def gs_bcast(l):
            return gsv[l]
        idxbufs = [ia, ib]
        rowbufs = [ra, rb]
        sems = [sa, sb]
        zero_f = jnp.float32(0.0)
        one_f = jnp.float32(1.0)

        def load_xyz(g):
            p3 = g * 48 + iota3
            x = plsc.load_gather(xyz, [p3])
            y = plsc.load_gather(xyz, [p3 + 1])
            z = plsc.load_gather(xyz, [p3 + 2])
            return x, y, z

        def corner_hashes(x, y, z, gsl):
            xi, yi, zi = corner_idx(x, y, z, gsl)
            hx = (xi, xi + 1)
            y0 = yi * jnp.int32(PR1)
            hy = (y0, y0 + jnp.int32(PR1))
            z0 = zi * jnp.int32(PR2)
            hz = (z0, z0 + jnp.int32(PR2))
            hs = []
            for c in range(8):
                i, j, k = c >> 2, (c >> 1) & 1, c & 1
                hs.append((hx[i] ^ hy[j] ^ hz[k]) & MASK)
            return xi, yi, zi, hs

        def corner_idx(x, y, z, gsl):
            xc = jnp.minimum(jnp.maximum(x, zero_f), one_f)
            yc = jnp.minimum(jnp.maximum(y, zero_f), one_f)
            zc = jnp.minimum(jnp.maximum(z, zero_f), one_f)
            xi = (xc / gsl).astype(jnp.int32)
            yi = (yc / gsl).astype(jnp.int32)
            zi = (zc / gsl).astype(jnp.int32)
            return xi, yi, zi

        def phase_a(l, g):
            x, y, z = load_xyz(g)
            _, _, _, hs = corner_hashes(x, y, z, gs_bcast(l))
            pos0 = g * 128
            loff = l * TS
            dst = idxbufs[l % 2]
            for c in range(8):
                blk = lax.shift_right_logical(hs[c] + loff, 2)
                plsc.store_scatter(dst, [pos0 + lane8c[c]], blk)

        def phase_c(l, g):
            x, y, z = load_xyz(g)
            gsl = gs_bcast(l)
            xi, yi, zi, hs = corner_hashes(x, y, z, gsl)
            wx = (x - xi.astype(jnp.float32) * gsl) / gsl
            wy = (y - yi.astype(jnp.float32) * gsl) / gsl
            wz = (z - zi.astype(jnp.float32) * gsl) / gsl
            pos0 = g * 128
            rows = rowbufs[l % 2]
            prow = g * 16 + iota
            offs = [(hs[c] & 3) * 2 for c in range(8)]
            for f in range(N_FEATS):
                e = [plsc.load_gather(rows, [pos0 + lane8c[c], offs[c] + f])
                     for c in range(8)]
                c00 = e[0] * (one_f - wx) + e[4] * wx
                c01 = e[1] * (one_f - wx) + e[5] * wx
                c10 = e[2] * (one_f - wx) + e[6] * wx
                c11 = e[3] * (one_f - wx) + e[7] * wx
                c0 = c00 * (one_f - wy) + c10 * wy
                c1 = c01 * (one_f - wy) + c11 * wy
                r = c0 * (one_f - wz) + c1 * wz
                plsc.store_scatter(ob, [prow, jnp.full((16,), 2 * l + f, jnp.int32)], r)

        def chunk_body(ci, carry):
            cb = base_w + ci * CHUNK
            pltpu.sync_copy(xt_hbm.at[pl.ds(cb * 3, CHUNK * 3)], xyz)
            lax.fori_loop(0, G, lambda g, _: phase_a(0, g), None)
            cps = [None] * N_LEVELS
            cps[0] = pltpu.async_copy(ftab_hbm.at[idxbufs[0]], rowbufs[0], sems[0])
            for l in range(N_LEVELS):
                if l + 1 < N_LEVELS:
                    lax.fori_loop(0, G, lambda g, _, l=l: phase_a(l + 1, g), None)
                    cps[l + 1] = pltpu.async_copy(
                        ftab_hbm.at[idxbufs[(l + 1) % 2]],
                        rowbufs[(l + 1) % 2], sems[(l + 1) % 2])
                cps[l].wait()
                lax.fori_loop(0, G, lambda g, _, l=l: phase_c(l, g), None)
            pltpu.sync_copy(ob, out_hbm.at[pl.ds(cb, CHUNK)])
            return carry

        lax.fori_loop(0, NCHUNK, chunk_body, None)

    return body(xt, ftab, gs)


def kernel(x, tables):
    b = jnp.exp((jnp.log(jnp.float32(512.0)) - jnp.log(jnp.float32(16.0)))
                / (N_LEVELS - 1))
    res = jnp.stack([jnp.floor(jnp.float32(16.0) * b ** i)
                     for i in range(N_LEVELS)])
    gs = (jnp.float32(1.0) - jnp.float32(0.0)) / res
    gs_rep = jnp.tile(gs[:, None], (1, 16))
    xt = x.reshape(3 * N_POINTS)
    ftab = tables.reshape(N_LEVELS * TS * N_FEATS // 8, 8)
    out = _sc_embed(xt, ftab, gs_rep)
    keep_mask = x == jnp.maximum(jnp.minimum(x, 1.0), 0.0)
    keep_all = jnp.sum(keep_mask, axis=-1) == keep_mask.shape[-1]
    return out, keep_all
